# baseline (speedup 1.0000x reference)
import functools

import jax
import jax.numpy as jnp
from jax import lax
from jax.experimental import pallas as pl
from jax.experimental.pallas import tpu as pltpu
from jax.experimental.pallas import tpu_sc as plsc

NC = 2
NS = 16
LANES = 16
EW = 128


def _mesh():
    return plsc.VectorSubcoreMesh(core_axis_name="c", subcore_axis_name="s")


def _deg_kernel(srcr, dstr, etr, t16, *, n_pad, garbage):
    rows = srcr.shape[0]
    wrows = rows // (NC * NS)

    @functools.partial(
        pl.kernel,
        out_type=[
            jax.ShapeDtypeStruct((NC, n_pad), jnp.float32),
            jax.ShapeDtypeStruct((rows, EW), jnp.int32),
            jax.ShapeDtypeStruct((rows, EW), jnp.int32),
        ],
        mesh=_mesh(),
        scratch_types=[
            pltpu.VMEM((16, EW), jnp.int32),
            pltpu.VMEM((16, EW), jnp.int32),
            pltpu.VMEM((16, EW), jnp.float32),
            pltpu.VMEM((16, EW), jnp.float32),
            pltpu.VMEM((16, EW), jnp.int32),
            pltpu.VMEM((16, EW), jnp.int32),
            pltpu.VMEM((16,), jnp.float32),
            pltpu.VMEM((n_pad // NS,), jnp.float32),
            pltpu.VMEM_SHARED((n_pad,), jnp.float32),
            pltpu.SemaphoreType.DMA,
        ],
    )
    def k(src_h, dst_h, et_h, t_h, degp_h, srcq_h, dstq_h,
          sbuf, dbuf, ebuf, mq, sq, dq, tbuf, zb, deg_sp, sem):
        c = lax.axis_index("c")
        s = lax.axis_index("s")
        w = c * NS + s
        seg = n_pad // NS

        pltpu.sync_copy(t_h, tbuf)
        tv = tbuf[...]

        @pl.loop(0, seg, step=LANES)
        def _(i):
            zb[pl.ds(i, LANES)] = jnp.zeros((LANES,), jnp.float32)

        pltpu.sync_copy(zb, deg_sp.at[pl.ds(s * seg, seg)])
        plsc.subcore_barrier()

        @pl.loop(0, wrows // 16)
        def _(bk):
            base = w * wrows + bk * 16
            pltpu.sync_copy(src_h.at[pl.ds(base, 16)], sbuf)
            pltpu.sync_copy(dst_h.at[pl.ds(base, 16)], dbuf)
            pltpu.sync_copy(et_h.at[pl.ds(base, 16)], ebuf)

            @pl.loop(0, 16)
            def _(r):
                @pl.loop(0, EW, step=LANES)
                def _(cc):
                    sv = sbuf[r, pl.ds(cc, LANES)]
                    dv = dbuf[r, pl.ds(cc, LANES)]
                    ev = ebuf[r, pl.ds(cc, LANES)]
                    m = (ev <= tv) & (sv != dv)
                    garb = garbage + lax.iota(jnp.int32, LANES)
                    mq[r, pl.ds(cc, LANES)] = jnp.where(m, 1.0, 0.0).astype(jnp.float32)
                    sq[r, pl.ds(cc, LANES)] = jnp.where(m, sv, garb)
                    dq[r, pl.ds(cc, LANES)] = jnp.where(m, dv, garb)

            pltpu.sync_copy(sq, srcq_h.at[pl.ds(base, 16)])
            pltpu.sync_copy(dq, dstq_h.at[pl.ds(base, 16)])

            @pl.loop(0, 16)
            def _(r):
                pltpu.async_copy(mq.at[r], deg_sp.at[sbuf.at[r]], sem, add=True)
                pltpu.async_copy(mq.at[r], deg_sp.at[dbuf.at[r]], sem, add=True)

            @pl.loop(0, 16)
            def _(r):
                pltpu.make_async_copy(mq.at[r], deg_sp.at[sbuf.at[r]], sem).wait()
                pltpu.make_async_copy(mq.at[r], deg_sp.at[dbuf.at[r]], sem).wait()

        plsc.subcore_barrier()
        pltpu.sync_copy(deg_sp.at[pl.ds(s * seg, seg)],
                        degp_h.at[c, pl.ds(s * seg, seg)])

    return k(srcr, dstr, etr, t16)


NBUF = 4
IBLK = 40


def _zero_acc(s, zb, acc, n_pad, sem):
    @pl.loop(0, 16)
    def _(r):
        @pl.loop(0, 128, step=LANES)
        def _(cc):
            zb[r, pl.ds(cc, LANES)] = jnp.zeros((LANES,), jnp.float32)

    aseg = n_pad // NS

    @pl.loop(0, aseg, step=16)
    def _(rr):
        pltpu.async_copy(zb, acc.at[pl.ds(s * aseg + rr, 16)], sem)

    @pl.loop(0, aseg, step=16)
    def _(rr):
        pltpu.make_async_copy(zb, acc.at[pl.ds(s * aseg + rr, 16)], sem).wait()


def _pipe_dir(get_src, acc, gsrc_h, ssrc_h, row0, nrows,
              gidx, sidx, bufs, gsems, ssems):
    bA, bB = bufs
    gA, gB = gsems
    sA, sB = ssems

    @pl.loop(0, nrows // IBLK)
    def _(bk):
        base = row0 + bk * IBLK
        pltpu.sync_copy(gsrc_h.at[pl.ds(base, IBLK)], gidx)
        pltpu.sync_copy(ssrc_h.at[pl.ds(base, IBLK)], sidx)
        pltpu.async_copy(get_src(gidx.at[0]), bA, gA)
        pltpu.async_copy(get_src(gidx.at[1]), bB, gB)

        @pl.loop(0, IBLK // 2 - 1)
        def _(k):
            a = 2 * k
            b = a + 1
            pltpu.make_async_copy(get_src(gidx.at[a]), bA, gA).wait()
            pltpu.async_copy(bA, acc.at[sidx.at[a]], sA, add=True)
            pltpu.make_async_copy(get_src(gidx.at[b]), bB, gB).wait()
            pltpu.async_copy(bB, acc.at[sidx.at[b]], sB, add=True)
            pltpu.make_async_copy(bA, acc.at[sidx.at[a]], sA).wait()
            pltpu.async_copy(get_src(gidx.at[a + 2]), bA, gA)
            pltpu.make_async_copy(bB, acc.at[sidx.at[b]], sB).wait()
            pltpu.async_copy(get_src(gidx.at[b + 2]), bB, gB)

        a = IBLK - 2
        b = IBLK - 1
        pltpu.make_async_copy(get_src(gidx.at[a]), bA, gA).wait()
        pltpu.async_copy(bA, acc.at[sidx.at[a]], sA, add=True)
        pltpu.make_async_copy(get_src(gidx.at[b]), bB, gB).wait()
        pltpu.async_copy(bB, acc.at[sidx.at[b]], sB, add=True)
        pltpu.make_async_copy(bA, acc.at[sidx.at[a]], sA).wait()
        pltpu.make_async_copy(bB, acc.at[sidx.at[b]], sB).wait()


def _agg_scratch(n_pad):
    return [
        pltpu.VMEM((IBLK, EW), jnp.int32),
        pltpu.VMEM((IBLK, EW), jnp.int32),
        pltpu.VMEM((EW, 128), jnp.float32),
        pltpu.VMEM((EW, 128), jnp.float32),
        pltpu.VMEM((16, 128), jnp.float32),
        pltpu.VMEM_SHARED((n_pad, 128), jnp.float32),
    ] + [pltpu.SemaphoreType.DMA] * 4


def _agg2_kernel(F, srcr, dstr, srcq, dstq, *, n_pad):
    d = F.shape[2]
    rows = srcr.shape[0]
    srows = rows // NS
    nseg = n_pad // NS

    @functools.partial(
        pl.kernel,
        out_type=jax.ShapeDtypeStruct((2, n_pad, d), jnp.float32),
        mesh=_mesh(),
        scratch_types=_agg_scratch(n_pad),
    )
    def k(F_h, src_h, dst_h, srcq_h, dstq_h, out_h, gidx, sidx,
          b0, b1, zb, acc, *sems):
        c = lax.axis_index("c")
        s = lax.axis_index("s")
        bufs = (b0, b1)
        gsems, ssems = sems[:2], sems[2:]

        _zero_acc(s, zb, acc, n_pad, gsems[0])
        plsc.subcore_barrier()

        get_src = lambda idx_ref: F_h.at[c].at[idx_ref]
        _pipe_dir(get_src, acc, src_h, dstq_h, s * srows, srows,
                  gidx, sidx, bufs, gsems, ssems)
        _pipe_dir(get_src, acc, dst_h, srcq_h, s * srows, srows,
                  gidx, sidx, bufs, gsems, ssems)

        plsc.subcore_barrier()
        pltpu.sync_copy(acc.at[pl.ds(s * nseg, nseg)],
                        out_h.at[c, pl.ds(s * nseg, nseg)])

    return k(F, srcr, dstr, srcq, dstq)


def _agg1_kernel(g, srcr, dstr, srcq, dstq, *, n_pad):
    rows = srcr.shape[0]
    wrows = rows // (NC * NS)
    nseg = n_pad // NS

    @functools.partial(
        pl.kernel,
        out_type=jax.ShapeDtypeStruct((2, n_pad, g.shape[1]), jnp.float32),
        mesh=_mesh(),
        scratch_types=_agg_scratch(n_pad),
    )
    def k(g_h, src_h, dst_h, srcq_h, dstq_h, out_h, gidx, sidx,
          b0, b1, zb, acc, *sems):
        c = lax.axis_index("c")
        s = lax.axis_index("s")
        w = c * NS + s
        bufs = (b0, b1)
        gsems, ssems = sems[:2], sems[2:]

        _zero_acc(s, zb, acc, n_pad, gsems[0])
        plsc.subcore_barrier()

        get_src = lambda idx_ref: g_h.at[idx_ref]
        _pipe_dir(get_src, acc, src_h, dstq_h, w * wrows, wrows,
                  gidx, sidx, bufs, gsems, ssems)
        _pipe_dir(get_src, acc, dst_h, srcq_h, w * wrows, wrows,
                  gidx, sidx, bufs, gsems, ssems)

        plsc.subcore_barrier()
        pltpu.sync_copy(acc.at[pl.ds(s * nseg, nseg)],
                        out_h.at[c, pl.ds(s * nseg, nseg)])

    return k(g, srcr, dstr, srcq, dstq)


def _tc_a(degp, x, h, *, rb):
    n, d = x.shape
    grid = -(-n // rb)

    def body(degp_ref, x_ref, h_ref, F_ref, nsb_ref):
        i = pl.program_id(0)
        dsum = (degp_ref[0, pl.ds(i * rb, rb)] + degp_ref[1, pl.ds(i * rb, rb)])
        ns = lax.rsqrt(jnp.maximum(dsum, 1.0))[:, None]
        nsb = jnp.broadcast_to(ns, (rb, d))
        nsb_ref[...] = nsb
        F_ref[0] = x_ref[...] * nsb
        F_ref[1] = h_ref[...] * nsb

    np_ = degp.shape[1]
    return pl.pallas_call(
        body,
        grid=(grid,),
        in_specs=[
            pl.BlockSpec((2, np_), lambda i: (0, 0)),
            pl.BlockSpec((rb, d), lambda i: (i, 0)),
            pl.BlockSpec((rb, d), lambda i: (i, 0)),
        ],
        out_specs=[
            pl.BlockSpec((2, rb, d), lambda i: (0, i, 0)),
            pl.BlockSpec((rb, d), lambda i: (i, 0)),
        ],
        out_shape=[
            jax.ShapeDtypeStruct((2, n, d), jnp.float32),
            jax.ShapeDtypeStruct((n, d), jnp.float32),
        ],
    )(degp, x, h)


def _tc_b(Agg, nsb, h, Wxr, bxr, Wxz, bxz, Wxh, bxh, Whr, bhr, Whz, bhz, *, rb):
    n, d = h.shape
    grid = -(-n // rb)

    def body(A_ref, nsb_ref, h_ref, wxr, cxr, wxz, cxz, wxh, cxh, whr, chr_,
             whz, chz, g_ref, z_ref, xh_ref):
        ns = nsb_ref[...]
        ax = A_ref[0] * ns
        ah = A_ref[1] * ns
        dot = functools.partial(jnp.dot, preferred_element_type=jnp.float32)
        xr = dot(ax, wxr[...]) + cxr[...]
        xz = dot(ax, wxz[...]) + cxz[...]
        xh = dot(ax, wxh[...]) + cxh[...]
        r = jax.nn.sigmoid(xr + dot(ah, whr[...]) + chr_[...])
        z = jax.nn.sigmoid(xz + dot(ah, whz[...]) + chz[...])
        g_ref[...] = r * h_ref[...] * ns
        z_ref[...] = z
        xh_ref[...] = xh

    wspec = pl.BlockSpec((d, d), lambda i: (0, 0))
    bspec = pl.BlockSpec((1, d), lambda i: (0, 0))
    rspec = pl.BlockSpec((rb, d), lambda i: (i, 0))
    return pl.pallas_call(
        body,
        grid=(grid,),
        in_specs=[pl.BlockSpec((2, rb, d), lambda i: (0, i, 0)),
                  rspec, rspec,
                  wspec, bspec, wspec, bspec, wspec, bspec,
                  wspec, bspec, wspec, bspec],
        out_specs=[rspec, rspec, rspec],
        out_shape=[jax.ShapeDtypeStruct((n, d), jnp.float32)] * 3,
    )(Agg, nsb, h, Wxr, bxr.reshape(1, d), Wxz, bxz.reshape(1, d),
      Wxh, bxh.reshape(1, d), Whr, bhr.reshape(1, d), Whz, bhz.reshape(1, d))


def _tc_c(P, nsb, xh, z, h, Whh, bhh, *, rb):
    n, d = h.shape
    grid = -(-n // rb)

    def body(P_ref, nsb_ref, xh_ref, z_ref, h_ref, whh, chh, out_ref):
        au = (P_ref[0] + P_ref[1]) * nsb_ref[...]
        u = jnp.tanh(xh_ref[...] +
                     jnp.dot(au, whh[...], preferred_element_type=jnp.float32) +
                     chh[...])
        dh = (1.0 - z_ref[...]) * (u - h_ref[...])
        nrm = jnp.sqrt(jnp.sum(dh * dh, axis=1, keepdims=True))
        out_ref[...] = dh / jnp.maximum(nrm, 1e-12)

    rspec = pl.BlockSpec((rb, d), lambda i: (i, 0))
    return pl.pallas_call(
        body,
        grid=(grid,),
        in_specs=[pl.BlockSpec((2, rb, d), lambda i: (0, i, 0)),
                  rspec, rspec, rspec, rspec,
                  pl.BlockSpec((d, d), lambda i: (0, 0)),
                  pl.BlockSpec((1, d), lambda i: (0, 0))],
        out_specs=rspec,
        out_shape=jax.ShapeDtypeStruct((n, d), jnp.float32),
    )(P, nsb, xh, z, h, Whh, bhh.reshape(1, d))


def kernel(t, h, x, edge_index, edge_t, Wxz, bxz, Wxr, bxr, Wxh, bxh,
           Whz, bhz, Whr, bhr, Whh, bhh):
    n, d = x.shape
    e = edge_index.shape[1]
    assert n % NS == 0 and d == 128

    wrows = -(-e // (EW * NC * NS))
    wrows = -(-wrows // 80) * 80
    rows = wrows * NC * NS
    ep = rows * EW
    pad = ep - e
    src0, dst0 = edge_index[0], edge_index[1]
    pad_i = jnp.arange(pad, dtype=jnp.int32)
    srcr = jnp.concatenate([src0, pad_i % n]).reshape(rows, EW)
    dstr = jnp.concatenate([dst0, (pad_i + 1) % n]).reshape(rows, EW)
    etr = jnp.concatenate([edge_t, jnp.full((pad,), jnp.inf, jnp.float32)]
                          ).reshape(rows, EW)
    t16 = jnp.broadcast_to(jnp.reshape(t, (1,)), (16,))

    n_pad = -(-(n + 1) // 256) * 256
    garbage = n

    degp, srcq, dstq = _deg_kernel(srcr, dstr, etr, t16,
                                   n_pad=n_pad, garbage=garbage)

    rb = 2048
    F, nsb = _tc_a(degp, x, h, rb=rb)

    Agg = _agg2_kernel(F, srcr, dstr, srcq, dstq, n_pad=n_pad)

    g, z, xh = _tc_b(Agg, nsb, h, Wxr, bxr, Wxz, bxz, Wxh, bxh,
                     Whr, bhr, Whz, bhz, rb=rb)

    P = _agg1_kernel(g, srcr, dstr, srcq, dstq, n_pad=n_pad)

    return _tc_c(P, nsb, xh, z, h, Whh, bhh, rb=rb)


# speedup vs baseline: 1.0121x; 1.0121x over previous
import functools

import jax
import jax.numpy as jnp
from jax import lax
from jax.experimental import pallas as pl
from jax.experimental.pallas import tpu as pltpu
from jax.experimental.pallas import tpu_sc as plsc

NC = 2
NS = 16
LANES = 16
EW = 128


def _mesh():
    return plsc.VectorSubcoreMesh(core_axis_name="c", subcore_axis_name="s")


def _deg_kernel(srcr, dstr, etr, t16, *, n_pad, garbage):
    rows = srcr.shape[0]
    wrows = rows // (NC * NS)

    @functools.partial(
        pl.kernel,
        out_type=[
            jax.ShapeDtypeStruct((NC, n_pad), jnp.float32),
            jax.ShapeDtypeStruct((rows, EW), jnp.int32),
            jax.ShapeDtypeStruct((rows, EW), jnp.int32),
        ],
        mesh=_mesh(),
        scratch_types=[
            pltpu.VMEM((16, EW), jnp.int32),
            pltpu.VMEM((16, EW), jnp.int32),
            pltpu.VMEM((16, EW), jnp.float32),
            pltpu.VMEM((16, EW), jnp.float32),
            pltpu.VMEM((16, EW), jnp.int32),
            pltpu.VMEM((16, EW), jnp.int32),
            pltpu.VMEM((16,), jnp.float32),
            pltpu.VMEM((n_pad // NS,), jnp.float32),
            pltpu.VMEM_SHARED((n_pad,), jnp.float32),
            pltpu.SemaphoreType.DMA,
        ],
    )
    def k(src_h, dst_h, et_h, t_h, degp_h, srcq_h, dstq_h,
          sbuf, dbuf, ebuf, mq, sq, dq, tbuf, zb, deg_sp, sem):
        c = lax.axis_index("c")
        s = lax.axis_index("s")
        w = c * NS + s
        seg = n_pad // NS

        pltpu.sync_copy(t_h, tbuf)
        tv = tbuf[...]

        @pl.loop(0, seg, step=LANES)
        def _(i):
            zb[pl.ds(i, LANES)] = jnp.zeros((LANES,), jnp.float32)

        pltpu.sync_copy(zb, deg_sp.at[pl.ds(s * seg, seg)])
        plsc.subcore_barrier()

        @pl.loop(0, wrows // 16)
        def _(bk):
            base = w * wrows + bk * 16
            pltpu.sync_copy(src_h.at[pl.ds(base, 16)], sbuf)
            pltpu.sync_copy(dst_h.at[pl.ds(base, 16)], dbuf)
            pltpu.sync_copy(et_h.at[pl.ds(base, 16)], ebuf)

            @pl.loop(0, 16)
            def _(r):
                @pl.loop(0, EW, step=LANES)
                def _(cc):
                    sv = sbuf[r, pl.ds(cc, LANES)]
                    dv = dbuf[r, pl.ds(cc, LANES)]
                    ev = ebuf[r, pl.ds(cc, LANES)]
                    m = (ev <= tv) & (sv != dv)
                    garb = garbage + lax.iota(jnp.int32, LANES)
                    mq[r, pl.ds(cc, LANES)] = jnp.where(m, 1.0, 0.0).astype(jnp.float32)
                    sq[r, pl.ds(cc, LANES)] = jnp.where(m, sv, garb)
                    dq[r, pl.ds(cc, LANES)] = jnp.where(m, dv, garb)

            pltpu.sync_copy(sq, srcq_h.at[pl.ds(base, 16)])
            pltpu.sync_copy(dq, dstq_h.at[pl.ds(base, 16)])

            @pl.loop(0, 16)
            def _(r):
                pltpu.async_copy(mq.at[r], deg_sp.at[sbuf.at[r]], sem, add=True)
                pltpu.async_copy(mq.at[r], deg_sp.at[dbuf.at[r]], sem, add=True)

            @pl.loop(0, 16)
            def _(r):
                pltpu.make_async_copy(mq.at[r], deg_sp.at[sbuf.at[r]], sem).wait()
                pltpu.make_async_copy(mq.at[r], deg_sp.at[dbuf.at[r]], sem).wait()

        plsc.subcore_barrier()
        pltpu.sync_copy(deg_sp.at[pl.ds(s * seg, seg)],
                        degp_h.at[c, pl.ds(s * seg, seg)])

    return k(srcr, dstr, etr, t16)


NBUF = 4
IBLK = 40


def _zero_acc(s, zb, acc, n_pad, sem):
    @pl.loop(0, 16)
    def _(r):
        @pl.loop(0, 128, step=LANES)
        def _(cc):
            zb[r, pl.ds(cc, LANES)] = jnp.zeros((LANES,), jnp.float32)

    aseg = n_pad // NS

    @pl.loop(0, aseg, step=16)
    def _(rr):
        pltpu.async_copy(zb, acc.at[pl.ds(s * aseg + rr, 16)], sem)

    @pl.loop(0, aseg, step=16)
    def _(rr):
        pltpu.make_async_copy(zb, acc.at[pl.ds(s * aseg + rr, 16)], sem).wait()


def _pipe_dir(get_src, acc, gsrc_h, ssrc_h, row0, nrows,
              gidx, sidx, bufs, gsems, ssems):
    bA, bB = bufs
    gA, gB = gsems
    sA, sB = ssems

    @pl.loop(0, nrows // IBLK)
    def _(bk):
        base = row0 + bk * IBLK
        pltpu.async_copy(gsrc_h.at[pl.ds(base, IBLK)], gidx, gA)
        pltpu.async_copy(ssrc_h.at[pl.ds(base, IBLK)], sidx, gB)
        pltpu.make_async_copy(gsrc_h.at[pl.ds(base, IBLK)], gidx, gA).wait()
        pltpu.make_async_copy(ssrc_h.at[pl.ds(base, IBLK)], sidx, gB).wait()
        pltpu.async_copy(get_src(gidx.at[0]), bA, gA)
        pltpu.async_copy(get_src(gidx.at[1]), bB, gB)

        @pl.loop(0, IBLK // 2 - 1)
        def _(k):
            a = 2 * k
            b = a + 1
            pltpu.make_async_copy(get_src(gidx.at[a]), bA, gA).wait()
            pltpu.async_copy(bA, acc.at[sidx.at[a]], sA, add=True)
            pltpu.make_async_copy(get_src(gidx.at[b]), bB, gB).wait()
            pltpu.async_copy(bB, acc.at[sidx.at[b]], sB, add=True)
            pltpu.make_async_copy(bA, acc.at[sidx.at[a]], sA).wait()
            pltpu.async_copy(get_src(gidx.at[a + 2]), bA, gA)
            pltpu.make_async_copy(bB, acc.at[sidx.at[b]], sB).wait()
            pltpu.async_copy(get_src(gidx.at[b + 2]), bB, gB)

        a = IBLK - 2
        b = IBLK - 1
        pltpu.make_async_copy(get_src(gidx.at[a]), bA, gA).wait()
        pltpu.async_copy(bA, acc.at[sidx.at[a]], sA, add=True)
        pltpu.make_async_copy(get_src(gidx.at[b]), bB, gB).wait()
        pltpu.async_copy(bB, acc.at[sidx.at[b]], sB, add=True)
        pltpu.make_async_copy(bA, acc.at[sidx.at[a]], sA).wait()
        pltpu.make_async_copy(bB, acc.at[sidx.at[b]], sB).wait()


def _agg_scratch(n_pad):
    return [
        pltpu.VMEM((IBLK, EW), jnp.int32),
        pltpu.VMEM((IBLK, EW), jnp.int32),
        pltpu.VMEM((EW, 128), jnp.float32),
        pltpu.VMEM((EW, 128), jnp.float32),
        pltpu.VMEM((16, 128), jnp.float32),
        pltpu.VMEM_SHARED((n_pad, 128), jnp.float32),
    ] + [pltpu.SemaphoreType.DMA] * 4


def _agg2_kernel(F, srcr, dstr, srcq, dstq, *, n_pad):
    d = F.shape[2]
    rows = srcr.shape[0]
    srows = rows // NS
    nseg = n_pad // NS

    @functools.partial(
        pl.kernel,
        out_type=jax.ShapeDtypeStruct((2, n_pad, d), jnp.float32),
        mesh=_mesh(),
        scratch_types=_agg_scratch(n_pad),
    )
    def k(F_h, src_h, dst_h, srcq_h, dstq_h, out_h, gidx, sidx,
          b0, b1, zb, acc, *sems):
        c = lax.axis_index("c")
        s = lax.axis_index("s")
        bufs = (b0, b1)
        gsems, ssems = sems[:2], sems[2:]

        _zero_acc(s, zb, acc, n_pad, gsems[0])
        plsc.subcore_barrier()

        get_src = lambda idx_ref: F_h.at[c].at[idx_ref]
        _pipe_dir(get_src, acc, src_h, dstq_h, s * srows, srows,
                  gidx, sidx, bufs, gsems, ssems)
        _pipe_dir(get_src, acc, dst_h, srcq_h, s * srows, srows,
                  gidx, sidx, bufs, gsems, ssems)

        plsc.subcore_barrier()
        pltpu.sync_copy(acc.at[pl.ds(s * nseg, nseg)],
                        out_h.at[c, pl.ds(s * nseg, nseg)])

    return k(F, srcr, dstr, srcq, dstq)


def _agg1_kernel(g, srcr, dstr, srcq, dstq, *, n_pad):
    rows = srcr.shape[0]
    wrows = rows // (NC * NS)
    nseg = n_pad // NS

    @functools.partial(
        pl.kernel,
        out_type=jax.ShapeDtypeStruct((2, n_pad, g.shape[1]), jnp.float32),
        mesh=_mesh(),
        scratch_types=_agg_scratch(n_pad),
    )
    def k(g_h, src_h, dst_h, srcq_h, dstq_h, out_h, gidx, sidx,
          b0, b1, zb, acc, *sems):
        c = lax.axis_index("c")
        s = lax.axis_index("s")
        w = c * NS + s
        bufs = (b0, b1)
        gsems, ssems = sems[:2], sems[2:]

        _zero_acc(s, zb, acc, n_pad, gsems[0])
        plsc.subcore_barrier()

        get_src = lambda idx_ref: g_h.at[idx_ref]
        _pipe_dir(get_src, acc, src_h, dstq_h, w * wrows, wrows,
                  gidx, sidx, bufs, gsems, ssems)
        _pipe_dir(get_src, acc, dst_h, srcq_h, w * wrows, wrows,
                  gidx, sidx, bufs, gsems, ssems)

        plsc.subcore_barrier()
        pltpu.sync_copy(acc.at[pl.ds(s * nseg, nseg)],
                        out_h.at[c, pl.ds(s * nseg, nseg)])

    return k(g, srcr, dstr, srcq, dstq)


def _tc_a(degp, x, h, *, rb):
    n, d = x.shape
    grid = -(-n // rb)

    def body(degp_ref, x_ref, h_ref, F_ref, nsb_ref):
        i = pl.program_id(0)
        dsum = (degp_ref[0, pl.ds(i * rb, rb)] + degp_ref[1, pl.ds(i * rb, rb)])
        ns = lax.rsqrt(jnp.maximum(dsum, 1.0))[:, None]
        nsb = jnp.broadcast_to(ns, (rb, d))
        nsb_ref[...] = nsb
        F_ref[0] = x_ref[...] * nsb
        F_ref[1] = h_ref[...] * nsb

    np_ = degp.shape[1]
    return pl.pallas_call(
        body,
        grid=(grid,),
        in_specs=[
            pl.BlockSpec((2, np_), lambda i: (0, 0)),
            pl.BlockSpec((rb, d), lambda i: (i, 0)),
            pl.BlockSpec((rb, d), lambda i: (i, 0)),
        ],
        out_specs=[
            pl.BlockSpec((2, rb, d), lambda i: (0, i, 0)),
            pl.BlockSpec((rb, d), lambda i: (i, 0)),
        ],
        out_shape=[
            jax.ShapeDtypeStruct((2, n, d), jnp.float32),
            jax.ShapeDtypeStruct((n, d), jnp.float32),
        ],
    )(degp, x, h)


def _tc_b(Agg, nsb, h, Wxr, bxr, Wxz, bxz, Wxh, bxh, Whr, bhr, Whz, bhz, *, rb):
    n, d = h.shape
    grid = -(-n // rb)

    def body(A_ref, nsb_ref, h_ref, wxr, cxr, wxz, cxz, wxh, cxh, whr, chr_,
             whz, chz, g_ref, z_ref, xh_ref):
        ns = nsb_ref[...]
        ax = A_ref[0] * ns
        ah = A_ref[1] * ns
        dot = functools.partial(jnp.dot, preferred_element_type=jnp.float32)
        xr = dot(ax, wxr[...]) + cxr[...]
        xz = dot(ax, wxz[...]) + cxz[...]
        xh = dot(ax, wxh[...]) + cxh[...]
        r = jax.nn.sigmoid(xr + dot(ah, whr[...]) + chr_[...])
        z = jax.nn.sigmoid(xz + dot(ah, whz[...]) + chz[...])
        g_ref[...] = r * h_ref[...] * ns
        z_ref[...] = z
        xh_ref[...] = xh

    wspec = pl.BlockSpec((d, d), lambda i: (0, 0))
    bspec = pl.BlockSpec((1, d), lambda i: (0, 0))
    rspec = pl.BlockSpec((rb, d), lambda i: (i, 0))
    return pl.pallas_call(
        body,
        grid=(grid,),
        in_specs=[pl.BlockSpec((2, rb, d), lambda i: (0, i, 0)),
                  rspec, rspec,
                  wspec, bspec, wspec, bspec, wspec, bspec,
                  wspec, bspec, wspec, bspec],
        out_specs=[rspec, rspec, rspec],
        out_shape=[jax.ShapeDtypeStruct((n, d), jnp.float32)] * 3,
    )(Agg, nsb, h, Wxr, bxr.reshape(1, d), Wxz, bxz.reshape(1, d),
      Wxh, bxh.reshape(1, d), Whr, bhr.reshape(1, d), Whz, bhz.reshape(1, d))


def _tc_c(P, nsb, xh, z, h, Whh, bhh, *, rb):
    n, d = h.shape
    grid = -(-n // rb)

    def body(P_ref, nsb_ref, xh_ref, z_ref, h_ref, whh, chh, out_ref):
        au = (P_ref[0] + P_ref[1]) * nsb_ref[...]
        u = jnp.tanh(xh_ref[...] +
                     jnp.dot(au, whh[...], preferred_element_type=jnp.float32) +
                     chh[...])
        dh = (1.0 - z_ref[...]) * (u - h_ref[...])
        nrm = jnp.sqrt(jnp.sum(dh * dh, axis=1, keepdims=True))
        out_ref[...] = dh / jnp.maximum(nrm, 1e-12)

    rspec = pl.BlockSpec((rb, d), lambda i: (i, 0))
    return pl.pallas_call(
        body,
        grid=(grid,),
        in_specs=[pl.BlockSpec((2, rb, d), lambda i: (0, i, 0)),
                  rspec, rspec, rspec, rspec,
                  pl.BlockSpec((d, d), lambda i: (0, 0)),
                  pl.BlockSpec((1, d), lambda i: (0, 0))],
        out_specs=rspec,
        out_shape=jax.ShapeDtypeStruct((n, d), jnp.float32),
    )(P, nsb, xh, z, h, Whh, bhh.reshape(1, d))


def kernel(t, h, x, edge_index, edge_t, Wxz, bxz, Wxr, bxr, Wxh, bxh,
           Whz, bhz, Whr, bhr, Whh, bhh):
    n, d = x.shape
    e = edge_index.shape[1]
    assert n % NS == 0 and d == 128

    wrows = -(-e // (EW * NC * NS))
    wrows = -(-wrows // 80) * 80
    rows = wrows * NC * NS
    ep = rows * EW
    pad = ep - e
    src0, dst0 = edge_index[0], edge_index[1]
    pad_i = jnp.arange(pad, dtype=jnp.int32)
    srcr = jnp.concatenate([src0, pad_i % n]).reshape(rows, EW)
    dstr = jnp.concatenate([dst0, (pad_i + 1) % n]).reshape(rows, EW)
    etr = jnp.concatenate([edge_t, jnp.full((pad,), jnp.inf, jnp.float32)]
                          ).reshape(rows, EW)
    t16 = jnp.broadcast_to(jnp.reshape(t, (1,)), (16,))

    n_pad = -(-(n + 1) // 256) * 256
    garbage = n

    degp, srcq, dstq = _deg_kernel(srcr, dstr, etr, t16,
                                   n_pad=n_pad, garbage=garbage)

    rb = 2048
    F, nsb = _tc_a(degp, x, h, rb=rb)

    Agg = _agg2_kernel(F, srcr, dstr, srcq, dstq, n_pad=n_pad)

    g, z, xh = _tc_b(Agg, nsb, h, Wxr, bxr, Wxz, bxz, Wxh, bxh,
                     Whr, bhr, Whz, bhz, rb=rb)

    P = _agg1_kernel(g, srcr, dstr, srcq, dstq, n_pad=n_pad)

    return _tc_c(P, nsb, xh, z, h, Whh, bhh, rb=rb)


# speedup vs baseline: 1.0141x; 1.0020x over previous
import functools

import jax
import jax.numpy as jnp
from jax import lax
from jax.experimental import pallas as pl
from jax.experimental.pallas import tpu as pltpu
from jax.experimental.pallas import tpu_sc as plsc

NC = 2
NS = 16
LANES = 16
EW = 128


def _mesh():
    return plsc.VectorSubcoreMesh(core_axis_name="c", subcore_axis_name="s")


def _deg_kernel(srcr, dstr, etr, t16, *, n_pad, garbage):
    rows = srcr.shape[0]
    wrows = rows // (NC * NS)

    @functools.partial(
        pl.kernel,
        out_type=[
            jax.ShapeDtypeStruct((NC, n_pad), jnp.float32),
            jax.ShapeDtypeStruct((rows, EW), jnp.int32),
            jax.ShapeDtypeStruct((rows, EW), jnp.int32),
        ],
        mesh=_mesh(),
        scratch_types=[
            pltpu.VMEM((16, EW), jnp.int32),
            pltpu.VMEM((16, EW), jnp.int32),
            pltpu.VMEM((16, EW), jnp.float32),
            pltpu.VMEM((16, EW), jnp.float32),
            pltpu.VMEM((16, EW), jnp.int32),
            pltpu.VMEM((16, EW), jnp.int32),
            pltpu.VMEM((16,), jnp.float32),
            pltpu.VMEM((n_pad // NS,), jnp.float32),
            pltpu.VMEM_SHARED((n_pad,), jnp.float32),
            pltpu.SemaphoreType.DMA,
        ],
    )
    def k(src_h, dst_h, et_h, t_h, degp_h, srcq_h, dstq_h,
          sbuf, dbuf, ebuf, mq, sq, dq, tbuf, zb, deg_sp, sem):
        c = lax.axis_index("c")
        s = lax.axis_index("s")
        w = c * NS + s
        seg = n_pad // NS

        pltpu.sync_copy(t_h, tbuf)
        tv = tbuf[...]

        @pl.loop(0, seg, step=LANES)
        def _(i):
            zb[pl.ds(i, LANES)] = jnp.zeros((LANES,), jnp.float32)

        pltpu.sync_copy(zb, deg_sp.at[pl.ds(s * seg, seg)])
        plsc.subcore_barrier()

        @pl.loop(0, wrows // 16)
        def _(bk):
            base = w * wrows + bk * 16
            pltpu.sync_copy(src_h.at[pl.ds(base, 16)], sbuf)
            pltpu.sync_copy(dst_h.at[pl.ds(base, 16)], dbuf)
            pltpu.sync_copy(et_h.at[pl.ds(base, 16)], ebuf)

            @pl.loop(0, 16)
            def _(r):
                @pl.loop(0, EW, step=LANES)
                def _(cc):
                    sv = sbuf[r, pl.ds(cc, LANES)]
                    dv = dbuf[r, pl.ds(cc, LANES)]
                    ev = ebuf[r, pl.ds(cc, LANES)]
                    m = (ev <= tv) & (sv != dv)
                    garb = garbage + lax.iota(jnp.int32, LANES)
                    mq[r, pl.ds(cc, LANES)] = jnp.where(m, 1.0, 0.0).astype(jnp.float32)
                    sq[r, pl.ds(cc, LANES)] = jnp.where(m, sv, garb)
                    dq[r, pl.ds(cc, LANES)] = jnp.where(m, dv, garb)

            pltpu.sync_copy(sq, srcq_h.at[pl.ds(base, 16)])
            pltpu.sync_copy(dq, dstq_h.at[pl.ds(base, 16)])

            @pl.loop(0, 16)
            def _(r):
                pltpu.async_copy(mq.at[r], deg_sp.at[sbuf.at[r]], sem, add=True)
                pltpu.async_copy(mq.at[r], deg_sp.at[dbuf.at[r]], sem, add=True)

            @pl.loop(0, 16)
            def _(r):
                pltpu.make_async_copy(mq.at[r], deg_sp.at[sbuf.at[r]], sem).wait()
                pltpu.make_async_copy(mq.at[r], deg_sp.at[dbuf.at[r]], sem).wait()

        plsc.subcore_barrier()
        pltpu.sync_copy(deg_sp.at[pl.ds(s * seg, seg)],
                        degp_h.at[c, pl.ds(s * seg, seg)])

    return k(srcr, dstr, etr, t16)


IBLK = 40


def _zero_acc(s, zb, acc, n_pad, sem):
    @pl.loop(0, 16)
    def _(r):
        @pl.loop(0, 128, step=LANES)
        def _(cc):
            zb[r, pl.ds(cc, LANES)] = jnp.zeros((LANES,), jnp.float32)

    aseg = n_pad // NS

    @pl.loop(0, aseg, step=16)
    def _(rr):
        pltpu.async_copy(zb, acc.at[pl.ds(s * aseg + rr, 16)], sem)

    @pl.loop(0, aseg, step=16)
    def _(rr):
        pltpu.make_async_copy(zb, acc.at[pl.ds(s * aseg + rr, 16)], sem).wait()


def _pipe_dir(get_src, acc, gsrc_h, ssrc_h, row0, nrows,
              gidx, sidx, bufs, gsems, ssems):
    bA, bB = bufs
    gA, gB = gsems
    sA, sB = ssems

    @pl.loop(0, nrows // IBLK)
    def _(bk):
        base = row0 + bk * IBLK
        pltpu.async_copy(gsrc_h.at[pl.ds(base, IBLK)], gidx, gA)
        pltpu.async_copy(ssrc_h.at[pl.ds(base, IBLK)], sidx, gB)
        pltpu.make_async_copy(gsrc_h.at[pl.ds(base, IBLK)], gidx, gA).wait()
        pltpu.make_async_copy(ssrc_h.at[pl.ds(base, IBLK)], sidx, gB).wait()
        pltpu.async_copy(get_src(gidx.at[0]), bA, gA)
        pltpu.async_copy(get_src(gidx.at[1]), bB, gB)

        @pl.loop(0, IBLK // 2 - 1)
        def _(k):
            a = 2 * k
            b = a + 1
            pltpu.make_async_copy(get_src(gidx.at[a]), bA, gA).wait()
            pltpu.async_copy(bA, acc.at[sidx.at[a]], sA, add=True)
            pltpu.make_async_copy(get_src(gidx.at[b]), bB, gB).wait()
            pltpu.async_copy(bB, acc.at[sidx.at[b]], sB, add=True)
            pltpu.make_async_copy(bA, acc.at[sidx.at[a]], sA).wait()
            pltpu.async_copy(get_src(gidx.at[a + 2]), bA, gA)
            pltpu.make_async_copy(bB, acc.at[sidx.at[b]], sB).wait()
            pltpu.async_copy(get_src(gidx.at[b + 2]), bB, gB)

        a = IBLK - 2
        b = IBLK - 1
        pltpu.make_async_copy(get_src(gidx.at[a]), bA, gA).wait()
        pltpu.async_copy(bA, acc.at[sidx.at[a]], sA, add=True)
        pltpu.make_async_copy(get_src(gidx.at[b]), bB, gB).wait()
        pltpu.async_copy(bB, acc.at[sidx.at[b]], sB, add=True)
        pltpu.make_async_copy(bA, acc.at[sidx.at[a]], sA).wait()
        pltpu.make_async_copy(bB, acc.at[sidx.at[b]], sB).wait()


def _agg_scratch(n_pad):
    return [
        pltpu.VMEM((IBLK, EW), jnp.int32),
        pltpu.VMEM((IBLK, EW), jnp.int32),
        pltpu.VMEM((EW, 128), jnp.float32),
        pltpu.VMEM((EW, 128), jnp.float32),
        pltpu.VMEM((16, 128), jnp.float32),
        pltpu.VMEM_SHARED((n_pad, 128), jnp.float32),
    ] + [pltpu.SemaphoreType.DMA] * 4


def _agg2_kernel(F, srcr, dstr, srcq, dstq, *, n_pad):
    d = F.shape[2]
    rows = srcr.shape[0]
    srows = rows // NS
    nseg = n_pad // NS

    @functools.partial(
        pl.kernel,
        out_type=jax.ShapeDtypeStruct((2, n_pad, d), jnp.float32),
        mesh=_mesh(),
        scratch_types=_agg_scratch(n_pad),
    )
    def k(F_h, src_h, dst_h, srcq_h, dstq_h, out_h, gidx, sidx,
          b0, b1, zb, acc, *sems):
        c = lax.axis_index("c")
        s = lax.axis_index("s")
        bufs = (b0, b1)
        gsems, ssems = sems[:2], sems[2:]

        _zero_acc(s, zb, acc, n_pad, gsems[0])
        plsc.subcore_barrier()

        get_src = lambda idx_ref: F_h.at[c].at[idx_ref]
        _pipe_dir(get_src, acc, src_h, dstq_h, s * srows, srows,
                  gidx, sidx, bufs, gsems, ssems)
        _pipe_dir(get_src, acc, dst_h, srcq_h, s * srows, srows,
                  gidx, sidx, bufs, gsems, ssems)

        plsc.subcore_barrier()
        pltpu.sync_copy(acc.at[pl.ds(s * nseg, nseg)],
                        out_h.at[c, pl.ds(s * nseg, nseg)])

    return k(F, srcr, dstr, srcq, dstq)


def _agg1_kernel(g, srcr, dstr, srcq, dstq, *, n_pad):
    rows = srcr.shape[0]
    wrows = rows // (NC * NS)
    nseg = n_pad // NS

    @functools.partial(
        pl.kernel,
        out_type=jax.ShapeDtypeStruct((2, n_pad, g.shape[1]), jnp.float32),
        mesh=_mesh(),
        scratch_types=_agg_scratch(n_pad),
    )
    def k(g_h, src_h, dst_h, srcq_h, dstq_h, out_h, gidx, sidx,
          b0, b1, zb, acc, *sems):
        c = lax.axis_index("c")
        s = lax.axis_index("s")
        w = c * NS + s
        bufs = (b0, b1)
        gsems, ssems = sems[:2], sems[2:]

        _zero_acc(s, zb, acc, n_pad, gsems[0])
        plsc.subcore_barrier()

        get_src = lambda idx_ref: g_h.at[idx_ref]
        _pipe_dir(get_src, acc, src_h, dstq_h, w * wrows, wrows,
                  gidx, sidx, bufs, gsems, ssems)
        _pipe_dir(get_src, acc, dst_h, srcq_h, w * wrows, wrows,
                  gidx, sidx, bufs, gsems, ssems)

        plsc.subcore_barrier()
        pltpu.sync_copy(acc.at[pl.ds(s * nseg, nseg)],
                        out_h.at[c, pl.ds(s * nseg, nseg)])

    return k(g, srcr, dstr, srcq, dstq)


def _tc_a(degp, x, h, *, rb):
    n, d = x.shape
    grid = -(-n // rb)

    def body(degp_ref, x_ref, h_ref, F_ref, nsb_ref):
        i = pl.program_id(0)
        dsum = (degp_ref[0, pl.ds(i * rb, rb)] + degp_ref[1, pl.ds(i * rb, rb)])
        ns = lax.rsqrt(jnp.maximum(dsum, 1.0))[:, None]
        nsb = jnp.broadcast_to(ns, (rb, d))
        nsb_ref[...] = nsb
        F_ref[0] = x_ref[...] * nsb
        F_ref[1] = h_ref[...] * nsb

    np_ = degp.shape[1]
    return pl.pallas_call(
        body,
        grid=(grid,),
        in_specs=[
            pl.BlockSpec((2, np_), lambda i: (0, 0)),
            pl.BlockSpec((rb, d), lambda i: (i, 0)),
            pl.BlockSpec((rb, d), lambda i: (i, 0)),
        ],
        out_specs=[
            pl.BlockSpec((2, rb, d), lambda i: (0, i, 0)),
            pl.BlockSpec((rb, d), lambda i: (i, 0)),
        ],
        out_shape=[
            jax.ShapeDtypeStruct((2, n, d), jnp.float32),
            jax.ShapeDtypeStruct((n, d), jnp.float32),
        ],
    )(degp, x, h)


def _tc_b(Agg, nsb, h, Wxr, bxr, Wxz, bxz, Wxh, bxh, Whr, bhr, Whz, bhz, *, rb):
    n, d = h.shape
    grid = -(-n // rb)

    def body(A_ref, nsb_ref, h_ref, wxr, cxr, wxz, cxz, wxh, cxh, whr, chr_,
             whz, chz, g_ref, z_ref, xh_ref):
        ns = nsb_ref[...]
        ax = A_ref[0] * ns
        ah = A_ref[1] * ns
        dot = functools.partial(jnp.dot, preferred_element_type=jnp.float32)
        xr = dot(ax, wxr[...]) + cxr[...]
        xz = dot(ax, wxz[...]) + cxz[...]
        xh = dot(ax, wxh[...]) + cxh[...]
        r = jax.nn.sigmoid(xr + dot(ah, whr[...]) + chr_[...])
        z = jax.nn.sigmoid(xz + dot(ah, whz[...]) + chz[...])
        g_ref[...] = r * h_ref[...] * ns
        z_ref[...] = z
        xh_ref[...] = xh

    wspec = pl.BlockSpec((d, d), lambda i: (0, 0))
    bspec = pl.BlockSpec((1, d), lambda i: (0, 0))
    rspec = pl.BlockSpec((rb, d), lambda i: (i, 0))
    return pl.pallas_call(
        body,
        grid=(grid,),
        in_specs=[pl.BlockSpec((2, rb, d), lambda i: (0, i, 0)),
                  rspec, rspec,
                  wspec, bspec, wspec, bspec, wspec, bspec,
                  wspec, bspec, wspec, bspec],
        out_specs=[rspec, rspec, rspec],
        out_shape=[jax.ShapeDtypeStruct((n, d), jnp.float32)] * 3,
    )(Agg, nsb, h, Wxr, bxr.reshape(1, d), Wxz, bxz.reshape(1, d),
      Wxh, bxh.reshape(1, d), Whr, bhr.reshape(1, d), Whz, bhz.reshape(1, d))


def _tc_c(P, nsb, xh, z, h, Whh, bhh, *, rb):
    n, d = h.shape
    grid = -(-n // rb)

    def body(P_ref, nsb_ref, xh_ref, z_ref, h_ref, whh, chh, out_ref):
        au = (P_ref[0] + P_ref[1]) * nsb_ref[...]
        u = jnp.tanh(xh_ref[...] +
                     jnp.dot(au, whh[...], preferred_element_type=jnp.float32) +
                     chh[...])
        dh = (1.0 - z_ref[...]) * (u - h_ref[...])
        nrm = jnp.sqrt(jnp.sum(dh * dh, axis=1, keepdims=True))
        out_ref[...] = dh / jnp.maximum(nrm, 1e-12)

    rspec = pl.BlockSpec((rb, d), lambda i: (i, 0))
    return pl.pallas_call(
        body,
        grid=(grid,),
        in_specs=[pl.BlockSpec((2, rb, d), lambda i: (0, i, 0)),
                  rspec, rspec, rspec, rspec,
                  pl.BlockSpec((d, d), lambda i: (0, 0)),
                  pl.BlockSpec((1, d), lambda i: (0, 0))],
        out_specs=rspec,
        out_shape=jax.ShapeDtypeStruct((n, d), jnp.float32),
    )(P, nsb, xh, z, h, Whh, bhh.reshape(1, d))


def kernel(t, h, x, edge_index, edge_t, Wxz, bxz, Wxr, bxr, Wxh, bxh,
           Whz, bhz, Whr, bhr, Whh, bhh):
    n, d = x.shape
    e = edge_index.shape[1]
    assert n % NS == 0 and d == 128

    wrows = -(-e // (EW * NC * NS))
    wrows = -(-wrows // 80) * 80
    rows = wrows * NC * NS
    ep = rows * EW
    pad = ep - e
    src0, dst0 = edge_index[0], edge_index[1]
    pad_i = jnp.arange(pad, dtype=jnp.int32)
    srcr = jnp.concatenate([src0, pad_i % n]).reshape(rows, EW)
    dstr = jnp.concatenate([dst0, (pad_i + 1) % n]).reshape(rows, EW)
    etr = jnp.concatenate([edge_t, jnp.full((pad,), jnp.inf, jnp.float32)]
                          ).reshape(rows, EW)
    t16 = jnp.broadcast_to(jnp.reshape(t, (1,)), (16,))

    n_pad = -(-(n + 1) // 256) * 256
    garbage = n

    degp, srcq, dstq = _deg_kernel(srcr, dstr, etr, t16,
                                   n_pad=n_pad, garbage=garbage)

    rb = 2048
    F, nsb = _tc_a(degp, x, h, rb=rb)

    Agg = _agg2_kernel(F, srcr, dstr, srcq, dstq, n_pad=n_pad)

    g, z, xh = _tc_b(Agg, nsb, h, Wxr, bxr, Wxz, bxz, Wxh, bxh,
                     Whr, bhr, Whz, bhz, rb=rb)

    P = _agg1_kernel(g, srcr, dstr, srcq, dstq, n_pad=n_pad)

    return _tc_c(P, nsb, xh, z, h, Whh, bhh, rb=rb)
